# baseline (speedup 1.0000x reference)
import jax
import jax.numpy as jnp
from jax import lax
from jax.experimental import pallas as pl
from jax.experimental.pallas import tpu as pltpu
from jax.experimental.pallas import tpu_sc as plsc

B, M, D = 1024, 200, 128
MP = 208
NCHUNK = MP // 16
ND = D // 16
NW = 32

B_SC = 672
B_TC = B - B_SC
RPW = B_SC // NW
NEG = -1e30

BB = 8


def _sc_body(comb_hbm, s0_hbm, s1_hbm, s2_hbm,
             out_p_hbm, out_l_hbm,
             comb_v, slot_v,
             lbuf_v, wbuf_v, stage_v,
             sem0, sem1, sem2, sem_out):
    cid = lax.axis_index("c")
    sid = lax.axis_index("s")
    wid = sid * 2 + cid
    base = B_TC + wid * RPW
    lane = lax.iota(jnp.int32, 16)
    zero16 = jnp.zeros((16,), jnp.float32)

    abase = (base // 8) * 8
    off = base - abase
    pltpu.sync_copy(comb_hbm.at[pl.ds(abase, 32)], comb_v)

    for k in range(3):
        for mrow in range(M, MP):
            for jd in range(ND):
                slot_v[k, mrow, pl.ds(jd * 16, 16)] = zero16

    s_hbms = (s0_hbm, s1_hbm, s2_hbm)
    sems = (sem0, sem1, sem2)

    def start_fill(k, row):
        pltpu.make_async_copy(
            s_hbms[k].at[row], slot_v.at[k, pl.ds(0, M)], sems[k]).start()

    def wait_fill(k):
        pltpu.make_async_copy(
            s_hbms[k].at[0], slot_v.at[k, pl.ds(0, M)], sems[k]).wait()

    for k in range(3):
        start_fill(k, base)

    def row_body(r, _):
        row = base + r
        orow = base - B_TC + r
        par = lax.rem(r, 2)
        rr = r + off
        u0 = tuple(comb_v[rr, pl.ds(MP + jd * 16, 16)] for jd in range(ND))
        wait_fill(0)
        wait_fill(1)
        wait_fill(2)

        def hop_body(h, u):
            def dots_chunk(c, _):
                lvec = zero16
                for j in range(16):
                    mrow = c * 16 + j
                    p0 = slot_v[h, mrow, pl.ds(0, 16)] * u[0]
                    p1 = slot_v[h, mrow, pl.ds(16, 16)] * u[1]
                    p2 = slot_v[h, mrow, pl.ds(32, 16)] * u[2]
                    p3 = slot_v[h, mrow, pl.ds(48, 16)] * u[3]
                    p4 = slot_v[h, mrow, pl.ds(64, 16)] * u[4]
                    p5 = slot_v[h, mrow, pl.ds(80, 16)] * u[5]
                    p6 = slot_v[h, mrow, pl.ds(96, 16)] * u[6]
                    p7 = slot_v[h, mrow, pl.ds(112, 16)] * u[7]
                    acc = ((p0 + p1) + (p2 + p3)) + ((p4 + p5) + (p6 + p7))
                    dsum = jnp.sum(acc)
                    lvec = jnp.where(lane == j, dsum, lvec)
                gpc = comb_v[rr, pl.ds(c * 16, 16)]
                lv = lvec * gpc
                valid = (c * 16 + lane) < M
                lbuf_v[pl.ds(c * 16, 16)] = jnp.where(valid, lv, NEG)
                return 0
            lax.fori_loop(0, NCHUNK, dots_chunk, 0)

            @pl.when(jnp.logical_and(h == 0, r + 1 < RPW))
            def _():
                start_fill(0, row + 1)

            @pl.when(jnp.logical_and(h == 1, r + 1 < RPW))
            def _():
                start_fill(1, row + 1)

            @pl.when(jnp.logical_and(h == 2, r + 1 < RPW))
            def _():
                start_fill(2, row + 1)

            def maxc(c, mx):
                return jnp.maximum(mx, lbuf_v[pl.ds(c * 16, 16)])
            mxv = lax.fori_loop(0, NCHUNK, maxc,
                                jnp.full((16,), NEG, jnp.float32))
            mx = jnp.max(mxv)

            def expc(c, s):
                e = jnp.exp(lbuf_v[pl.ds(c * 16, 16)] - mx)
                wbuf_v[pl.ds(c * 16, 16)] = e
                return s + e
            sv = lax.fori_loop(0, NCHUNK, expc, zero16)
            total = jnp.sum(sv)
            inv = jnp.ones((16,), jnp.float32) / jnp.broadcast_to(total, (16,))

            def pool():
                def pool_chunk(c, o):
                    wv = (wbuf_v[pl.ds(c * 16, 16)]
                          * comb_v[rr, pl.ds(c * 16, 16)])
                    for j in range(16):
                        mrow = c * 16 + j
                        ws = wv[j]
                        o = tuple(
                            o[jd] + ws * slot_v[h + 1, mrow,
                                                pl.ds(jd * 16, 16)]
                            for jd in range(ND))
                    return o
                o = lax.fori_loop(0, NCHUNK, pool_chunk,
                                  tuple(zero16 for _ in range(ND)))
                return tuple(u[jd] + o[jd] * inv for jd in range(ND))

            def last():
                def outc(c, _):
                    stage_v[par, 0, pl.ds(c * 16, 16)] = (
                        wbuf_v[pl.ds(c * 16, 16)] * inv)
                    stage_v[par, 1, pl.ds(c * 16, 16)] = (
                        lbuf_v[pl.ds(c * 16, 16)])
                    return 0
                lax.fori_loop(0, NCHUNK, outc, 0)
                return u

            return lax.cond(h < 2, pool, last)

        @pl.when(r >= 2)
        def _():
            pltpu.make_async_copy(
                stage_v.at[par, 0], out_p_hbm.at[orow], sem_out).wait()
            pltpu.make_async_copy(
                stage_v.at[par, 1], out_l_hbm.at[orow], sem_out).wait()

        lax.fori_loop(0, 3, hop_body, u0)

        pltpu.make_async_copy(
            stage_v.at[par, 0], out_p_hbm.at[orow], sem_out).start()
        pltpu.make_async_copy(
            stage_v.at[par, 1], out_l_hbm.at[orow], sem_out).start()
        return 0

    lax.fori_loop(0, RPW, row_body, 0)

    for _ in range(4):
        pltpu.make_async_copy(
            stage_v.at[0, 0], out_p_hbm.at[base - B_TC], sem_out).wait()


def _tc_body(q_ref, gp_ref, s0_ref, s1_ref, s2_ref, out_p_ref, out_l_ref):
    stories = (s0_ref, s1_ref, s2_ref)
    u = q_ref[...]
    gp = gp_ref[...]
    logits = None
    p = None
    for hop in range(3):
        sA = stories[hop][...]
        dots = jax.lax.dot_general(
            sA, u, (((2,), (1,)), ((0,), (0,))),
            preferred_element_type=jnp.float32)
        logits = dots * gp
        mx = jnp.max(logits, axis=1, keepdims=True)
        e = jnp.exp(logits - mx)
        p = e / jnp.sum(e, axis=1, keepdims=True)
        if hop < 2:
            sC = stories[hop + 1][...]
            w = p * gp
            o = jax.lax.dot_general(
                w, sC, (((1,), (1,)), ((0,), (0,))),
                preferred_element_type=jnp.float32)
            u = u + o
    out_p_ref[...] = p
    out_l_ref[...] = logits


@jax.jit
def _run(query_vector, comb, global_pointer,
         m_story_0, m_story_1, m_story_2):
    mesh = plsc.VectorSubcoreMesh(core_axis_name="c", subcore_axis_name="s")
    sc = pl.kernel(
        _sc_body,
        out_type=(
            jax.ShapeDtypeStruct((B_SC, MP), jnp.float32),
            jax.ShapeDtypeStruct((B_SC, MP), jnp.float32),
        ),
        mesh=mesh,
        compiler_params=pltpu.CompilerParams(needs_layout_passes=False),
        scratch_types=[
            pltpu.VMEM((32, MP + D), jnp.float32),
            pltpu.VMEM((3, MP, D), jnp.float32),
            pltpu.VMEM((MP,), jnp.float32),
            pltpu.VMEM((MP,), jnp.float32),
            pltpu.VMEM((2, 2, MP), jnp.float32),
            pltpu.SemaphoreType.DMA,
            pltpu.SemaphoreType.DMA,
            pltpu.SemaphoreType.DMA,
            pltpu.SemaphoreType.DMA,
        ],
    )
    sc_p, sc_l = sc(comb, m_story_0, m_story_1, m_story_2)

    story_spec = pl.BlockSpec((BB, M, D), lambda i: (i, 0, 0))
    vec_spec = pl.BlockSpec((BB, D), lambda i: (i, 0))
    m_spec = pl.BlockSpec((BB, M), lambda i: (i, 0))
    out_spec = pl.BlockSpec((BB, M), lambda i: (i, 0))
    tc_p, tc_l = pl.pallas_call(
        _tc_body,
        grid=(B_TC // BB,),
        in_specs=[vec_spec, m_spec, story_spec, story_spec, story_spec],
        out_specs=[out_spec, out_spec],
        out_shape=[
            jax.ShapeDtypeStruct((B_TC, M), jnp.float32),
            jax.ShapeDtypeStruct((B_TC, M), jnp.float32),
        ],
    )(query_vector, global_pointer, m_story_0, m_story_1, m_story_2)

    out_p = jnp.concatenate([tc_p, sc_p[:, :M]], axis=0)
    out_l = jnp.concatenate([tc_l, sc_l[:, :M]], axis=0)
    return out_p, out_l


def kernel(query_vector, global_pointer, m_story_0, m_story_1, m_story_2, m_story_3):
    del m_story_3
    gp_pad = jnp.pad(global_pointer, ((0, 0), (0, MP - M)))
    comb = jnp.pad(jnp.concatenate([gp_pad, query_vector], axis=1),
                   ((0, 8), (0, 0)))
    return _run(query_vector, comb, global_pointer,
                m_story_0, m_story_1, m_story_2)


# speedup vs baseline: 1.0337x; 1.0337x over previous
import jax
import jax.numpy as jnp
from jax import lax
from jax.experimental import pallas as pl
from jax.experimental.pallas import tpu as pltpu
from jax.experimental.pallas import tpu_sc as plsc

B, M, D = 1024, 200, 128
MP = 208
NCHUNK = MP // 16
ND = D // 16
NW = 32

B_SC = 672
B_TC = B - B_SC
RPW = B_SC // NW
NEG = -1e30

BB = 16


def _sc_body(comb_hbm, s0_hbm, s1_hbm, s2_hbm,
             out_p_hbm, out_l_hbm,
             comb_v, slot_v,
             lbuf_v, wbuf_v, stage_v,
             sem0, sem1, sem2, sem_out):
    cid = lax.axis_index("c")
    sid = lax.axis_index("s")
    wid = sid * 2 + cid
    base = B_TC + wid * RPW
    lane = lax.iota(jnp.int32, 16)
    zero16 = jnp.zeros((16,), jnp.float32)

    abase = (base // 8) * 8
    off = base - abase
    pltpu.sync_copy(comb_hbm.at[pl.ds(abase, 32)], comb_v)

    for k in range(3):
        for mrow in range(M, MP):
            for jd in range(ND):
                slot_v[k, mrow, pl.ds(jd * 16, 16)] = zero16

    s_hbms = (s0_hbm, s1_hbm, s2_hbm)
    sems = (sem0, sem1, sem2)

    def start_fill(k, row):
        pltpu.make_async_copy(
            s_hbms[k].at[row], slot_v.at[k, pl.ds(0, M)], sems[k]).start()

    def wait_fill(k):
        pltpu.make_async_copy(
            s_hbms[k].at[0], slot_v.at[k, pl.ds(0, M)], sems[k]).wait()

    for k in range(3):
        start_fill(k, base)

    def row_body(r, _):
        row = base + r
        orow = base - B_TC + r
        par = lax.rem(r, 2)
        rr = r + off
        u0 = tuple(comb_v[rr, pl.ds(MP + jd * 16, 16)] for jd in range(ND))
        wait_fill(0)
        wait_fill(1)
        wait_fill(2)

        def hop_body(h, u):
            def dots_chunk(c, _):
                lvec = zero16
                for j in range(16):
                    mrow = c * 16 + j
                    p0 = slot_v[h, mrow, pl.ds(0, 16)] * u[0]
                    p1 = slot_v[h, mrow, pl.ds(16, 16)] * u[1]
                    p2 = slot_v[h, mrow, pl.ds(32, 16)] * u[2]
                    p3 = slot_v[h, mrow, pl.ds(48, 16)] * u[3]
                    p4 = slot_v[h, mrow, pl.ds(64, 16)] * u[4]
                    p5 = slot_v[h, mrow, pl.ds(80, 16)] * u[5]
                    p6 = slot_v[h, mrow, pl.ds(96, 16)] * u[6]
                    p7 = slot_v[h, mrow, pl.ds(112, 16)] * u[7]
                    acc = ((p0 + p1) + (p2 + p3)) + ((p4 + p5) + (p6 + p7))
                    dsum = jnp.sum(acc)
                    lvec = jnp.where(lane == j, dsum, lvec)
                gpc = comb_v[rr, pl.ds(c * 16, 16)]
                lv = lvec * gpc
                valid = (c * 16 + lane) < M
                lbuf_v[pl.ds(c * 16, 16)] = jnp.where(valid, lv, NEG)
                return 0
            lax.fori_loop(0, NCHUNK, dots_chunk, 0)

            @pl.when(jnp.logical_and(h == 0, r + 1 < RPW))
            def _():
                start_fill(0, row + 1)

            @pl.when(jnp.logical_and(h == 1, r + 1 < RPW))
            def _():
                start_fill(1, row + 1)

            @pl.when(jnp.logical_and(h == 2, r + 1 < RPW))
            def _():
                start_fill(2, row + 1)

            def maxc(c, mx):
                return jnp.maximum(mx, lbuf_v[pl.ds(c * 16, 16)])
            mxv = lax.fori_loop(0, NCHUNK, maxc,
                                jnp.full((16,), NEG, jnp.float32))
            mx = jnp.max(mxv)

            def expc(c, s):
                e = jnp.exp(lbuf_v[pl.ds(c * 16, 16)] - mx)
                wbuf_v[pl.ds(c * 16, 16)] = e
                return s + e
            sv = lax.fori_loop(0, NCHUNK, expc, zero16)
            total = jnp.sum(sv)
            inv = jnp.ones((16,), jnp.float32) / jnp.broadcast_to(total, (16,))

            def pool():
                def pool_chunk(c, o):
                    wv = (wbuf_v[pl.ds(c * 16, 16)]
                          * comb_v[rr, pl.ds(c * 16, 16)])
                    for j in range(16):
                        mrow = c * 16 + j
                        ws = wv[j]
                        o = tuple(
                            o[jd] + ws * slot_v[h + 1, mrow,
                                                pl.ds(jd * 16, 16)]
                            for jd in range(ND))
                    return o
                o = lax.fori_loop(0, NCHUNK, pool_chunk,
                                  tuple(zero16 for _ in range(ND)))
                return tuple(u[jd] + o[jd] * inv for jd in range(ND))

            def last():
                def outc(c, _):
                    stage_v[par, 0, pl.ds(c * 16, 16)] = (
                        wbuf_v[pl.ds(c * 16, 16)] * inv)
                    stage_v[par, 1, pl.ds(c * 16, 16)] = (
                        lbuf_v[pl.ds(c * 16, 16)])
                    return 0
                lax.fori_loop(0, NCHUNK, outc, 0)
                return u

            return lax.cond(h < 2, pool, last)

        @pl.when(r >= 2)
        def _():
            pltpu.make_async_copy(
                stage_v.at[par, 0], out_p_hbm.at[orow], sem_out).wait()
            pltpu.make_async_copy(
                stage_v.at[par, 1], out_l_hbm.at[orow], sem_out).wait()

        lax.fori_loop(0, 3, hop_body, u0)

        pltpu.make_async_copy(
            stage_v.at[par, 0], out_p_hbm.at[orow], sem_out).start()
        pltpu.make_async_copy(
            stage_v.at[par, 1], out_l_hbm.at[orow], sem_out).start()
        return 0

    lax.fori_loop(0, RPW, row_body, 0)

    for _ in range(4):
        pltpu.make_async_copy(
            stage_v.at[0, 0], out_p_hbm.at[base - B_TC], sem_out).wait()


def _tc_body(q_ref, gp_ref, s0_ref, s1_ref, s2_ref, out_p_ref, out_l_ref):
    stories = (s0_ref, s1_ref, s2_ref)
    u = q_ref[...]
    gp = gp_ref[...]
    logits = None
    p = None
    for hop in range(3):
        sA = stories[hop][...]
        logits = jnp.sum(sA * u[:, None, :], axis=2) * gp
        mx = jnp.max(logits, axis=1, keepdims=True)
        e = jnp.exp(logits - mx)
        p = e / jnp.sum(e, axis=1, keepdims=True)
        if hop < 2:
            sC = stories[hop + 1][...]
            w = (p * gp)[:, :, None]
            u = u + jnp.sum(sC * w, axis=1)
    out_p_ref[...] = p
    out_l_ref[...] = logits


@jax.jit
def _run(query_vector, comb, global_pointer,
         m_story_0, m_story_1, m_story_2):
    mesh = plsc.VectorSubcoreMesh(core_axis_name="c", subcore_axis_name="s")
    sc = pl.kernel(
        _sc_body,
        out_type=(
            jax.ShapeDtypeStruct((B_SC, MP), jnp.float32),
            jax.ShapeDtypeStruct((B_SC, MP), jnp.float32),
        ),
        mesh=mesh,
        compiler_params=pltpu.CompilerParams(needs_layout_passes=False),
        scratch_types=[
            pltpu.VMEM((32, MP + D), jnp.float32),
            pltpu.VMEM((3, MP, D), jnp.float32),
            pltpu.VMEM((MP,), jnp.float32),
            pltpu.VMEM((MP,), jnp.float32),
            pltpu.VMEM((2, 2, MP), jnp.float32),
            pltpu.SemaphoreType.DMA,
            pltpu.SemaphoreType.DMA,
            pltpu.SemaphoreType.DMA,
            pltpu.SemaphoreType.DMA,
        ],
    )
    sc_p, sc_l = sc(comb, m_story_0, m_story_1, m_story_2)

    story_spec = pl.BlockSpec((BB, M, D), lambda i: (i, 0, 0))
    vec_spec = pl.BlockSpec((BB, D), lambda i: (i, 0))
    m_spec = pl.BlockSpec((BB, M), lambda i: (i, 0))
    out_spec = pl.BlockSpec((BB, M), lambda i: (i, 0))
    tc_p, tc_l = pl.pallas_call(
        _tc_body,
        grid=(B_TC // BB,),
        in_specs=[vec_spec, m_spec, story_spec, story_spec, story_spec],
        out_specs=[out_spec, out_spec],
        out_shape=[
            jax.ShapeDtypeStruct((B_TC, M), jnp.float32),
            jax.ShapeDtypeStruct((B_TC, M), jnp.float32),
        ],
    )(query_vector, global_pointer, m_story_0, m_story_1, m_story_2)

    out_p = jnp.concatenate([tc_p, sc_p[:, :M]], axis=0)
    out_l = jnp.concatenate([tc_l, sc_l[:, :M]], axis=0)
    return out_p, out_l


def kernel(query_vector, global_pointer, m_story_0, m_story_1, m_story_2, m_story_3):
    del m_story_3
    gp_pad = jnp.pad(global_pointer, ((0, 0), (0, MP - M)))
    comb = jnp.pad(jnp.concatenate([gp_pad, query_vector], axis=1),
                   ((0, 8), (0, 0)))
    return _run(query_vector, comb, global_pointer,
                m_story_0, m_story_1, m_story_2)
